# baseline (speedup 1.0000x reference)
import functools

import jax
import jax.numpy as jnp
from jax import lax
from jax.experimental import pallas as pl
from jax.experimental.pallas import tpu as pltpu
from jax.experimental.pallas import tpu_sc as plsc

B = 4096
HIST = 200
D = 16

NC = 2
NS = 16
NW = NC * NS

SEQ_N = B * HIST
SEQ_PER_W = SEQ_N // NW
CH = 1280
NCH = SEQ_PER_W // CH
NPAIR = NCH // 2
IPG = 128
GPC = CH // IPG
SMALL_PER_W = B // NW


BR = 2
BSTEPS = (B // NW) // BR
MROWS = 8
MSTEPS = MROWS // BR
EXN = BR * HIST


def _extract_body(seq3d, tus, tts, tis,
                  s0_o, s1_o, s2_o, mask_o,
                  u0_o, u1_o, u2_o, u3_o, u4_o,
                  ts0_o, ts1_o, ts2_o, ti0_o, ti1_o, ti2_o,
                  slab, c0, c1, c2, mk,
                  sl5, sl3, sc0, sc1, sc2, sc3, sc4,
                  ssem):
  wid = lax.axis_index("s") * NC + lax.axis_index("c")
  cols = [c0, c1, c2]
  outs = [s0_o, s1_o, s2_o]
  lane = lax.broadcasted_iota(jnp.int32, (16,), 0)

  def row0(st):
    return (wid * BSTEPS + st) * BR

  def step(st, _):
    pltpu.sync_copy(seq3d.at[pl.ds(row0(st), BR)], slab)

    def grp(g, _):
      q = g * 16 + lane
      i = q // HIST
      j = q - HIST * i
      for c in range(3):
        v = plsc.load_gather(slab, [i, j, jnp.full((16,), c, jnp.int32)])
        cols[c][pl.ds(g * 16, 16)] = v
      return 0
    lax.fori_loop(0, EXN // 16, grp, 0)

    half = (st % MSTEPS) * BR

    def mrow(r, _):
      for off in list(range(0, HIST - 16, 16)) + [HIST - 16]:
        v = c0[pl.ds(r * HIST + off, 16)]
        m = jnp.where(v != 0, jnp.int32(1), jnp.int32(0))
        plsc.store_scatter(
            mk, [jnp.full((16,), half + r, jnp.int32), off + lane], m)
      return 0
    lax.fori_loop(0, BR, mrow, 0)

    pos = row0(st) * HIST
    for c in range(3):
      pltpu.sync_copy(cols[c], outs[c].at[pl.ds(pos, EXN)])

    @pl.when(st % MSTEPS == MSTEPS - 1)
    def _():
      off = pl.multiple_of(row0(st) - (MROWS - BR), MROWS)
      pltpu.sync_copy(mk, mask_o.at[pl.ds(off, MROWS)])
    return 0

  lax.fori_loop(0, BSTEPS, step, 0)

  sbase = wid * SMALL_PER_W
  scols = [sc0, sc1, sc2, sc3, sc4]
  for src, slab, ncols, souts in (
      (tus, sl5, 5, [u0_o, u1_o, u2_o, u3_o, u4_o]),
      (tts, sl3, 3, [ts0_o, ts1_o, ts2_o]),
      (tis, sl3, 3, [ti0_o, ti1_o, ti2_o]),
  ):
    pltpu.async_copy(src.at[pl.ds(sbase, SMALL_PER_W)], slab, ssem).wait()
    def sgrp(g, _, slab=slab, ncols=ncols):
      i = g * 16 + lane
      for c in range(ncols):
        v = plsc.load_gather(slab, [i, jnp.full((16,), c, jnp.int32)])
        scols[c][pl.ds(g * 16, 16)] = v
      return 0
    lax.fori_loop(0, SMALL_PER_W // 16, sgrp, 0)
    for c in range(ncols):
      pltpu.sync_copy(scols[c], souts[c].at[pl.ds(sbase, SMALL_PER_W)])


_ex_call = functools.partial(
    pl.kernel,
    mesh=plsc.VectorSubcoreMesh(
        core_axis_name="c", subcore_axis_name="s", num_cores=NC),
    out_type=[
        jax.ShapeDtypeStruct((SEQ_N,), jnp.int32),
        jax.ShapeDtypeStruct((SEQ_N,), jnp.int32),
        jax.ShapeDtypeStruct((SEQ_N,), jnp.int32),
        jax.ShapeDtypeStruct((B, HIST), jnp.int32),
    ] + [jax.ShapeDtypeStruct((B,), jnp.int32)] * 11,
    scratch_types=[
        pltpu.VMEM((BR, HIST, 3), jnp.int32),
        pltpu.VMEM((EXN,), jnp.int32),
        pltpu.VMEM((EXN,), jnp.int32),
        pltpu.VMEM((EXN,), jnp.int32),
        pltpu.VMEM((MROWS, HIST), jnp.int32),
        pltpu.VMEM((SMALL_PER_W, 5), jnp.int32),
        pltpu.VMEM((SMALL_PER_W, 3), jnp.int32),
        pltpu.VMEM((SMALL_PER_W,), jnp.int32),
        pltpu.VMEM((SMALL_PER_W,), jnp.int32),
        pltpu.VMEM((SMALL_PER_W,), jnp.int32),
        pltpu.VMEM((SMALL_PER_W,), jnp.int32),
        pltpu.VMEM((SMALL_PER_W,), jnp.int32),
        pltpu.SemaphoreType.DMA,
    ],
    compiler_params=pltpu.CompilerParams(needs_layout_passes=False),
)(_extract_body)


def _sc_body(s0, s1, s2, wseq0, wseq1, wseq2,
             u0, u1, u2, u3, u4, wu0, wu1, wu2, wu3, wu4,
             ts0, ts1, ts2,
             ti0, ti1, ti2, wi0, wi1, wi2,
             user_o, seq_o, tseq_o, tside_o,
             idx0, idx1, rows0, rows1,
             sidx_v, srows_v, sbig_v,
             gsem, wsem0, wsem1, ssem):
  wid = lax.axis_index("s") * NC + lax.axis_index("c")
  base = wid * SEQ_PER_W

  sidx = [s0, s1, s2]
  wseq = [wseq0, wseq1, wseq2]
  idxs = [idx0, idx1]
  rows = [rows0, rows1]
  wsems = [wsem0, wsem1]

  def fire(t, p):
    def go(j, _):
      pltpu.async_copy(
          wseq[t].at[idxs[p].at[pl.ds(j * IPG, IPG)]],
          rows[p].at[pl.ds(j * IPG, IPG)], gsem)
      return 0
    lax.fori_loop(0, GPC, go, 0)

  def drain():
    def go(j, _):
      pltpu.make_async_copy(
          wseq[0].at[idx0.at[pl.ds(0, IPG)]],
          rows0.at[pl.ds(0, IPG)], gsem).wait()
      return 0
    lax.fori_loop(0, GPC, go, 0)

  def wait_write(p):
    pltpu.make_async_copy(
        rows[p], seq_o.at[pl.ds(0, CH), pl.ds(0, 16)], wsems[p]).wait()

  def pair_body(k, _):
    for half in range(2):
      c = 2 * k + half
      pos = base + c * CH
      for t in range(3):
        stage = 3 * half + t
        p = stage % 2
        pltpu.sync_copy(sidx[t].at[pl.ds(pos, CH)], idxs[p])
        if stage < 2:
          @pl.when(k > 0)
          def _():
            wait_write(p)
        else:
          wait_write(p)
        fire(t, p)
        drain()
        pltpu.async_copy(
            rows[p], seq_o.at[pl.ds(pos, CH), pl.ds(16 * t, 16)], wsems[p])
    return 0

  lax.fori_loop(0, NPAIR, pair_body, 0)
  wait_write(0)
  wait_write(1)

  sbase = wid * SMALL_PER_W
  lane = lax.broadcasted_iota(jnp.int32, (16,), 0)
  for idxs_g, tables, out, w in (
      ([u0, u1, u2, u3, u4], [wu0, wu1, wu2, wu3, wu4], user_o, 5 * D),
      ([ts0, ts1, ts2], wseq, tseq_o, 3 * D),
      ([ti0, ti1, ti2], [wi0, wi1, wi2], tside_o, 3 * D),
  ):
    for c, (idx_hbm, table) in enumerate(zip(idxs_g, tables)):
      pltpu.sync_copy(idx_hbm.at[pl.ds(sbase, SMALL_PER_W)], sidx_v)
      pltpu.async_copy(table.at[sidx_v], srows_v, ssem).wait()

      def interleave(r, _, c=c, w=w):
        v = plsc.load_gather(srows_v, [jnp.full((16,), r, jnp.int32), lane])
        sbig_v[pl.ds(r * w + 16 * c, 16)] = v
        return 0
      lax.fori_loop(0, SMALL_PER_W, interleave, 0)
    pltpu.sync_copy(
        sbig_v.at[pl.ds(0, SMALL_PER_W * w)],
        out.at[pl.ds(sbase * w, SMALL_PER_W * w)])


_sc_call = functools.partial(
    pl.kernel,
    mesh=plsc.VectorSubcoreMesh(
        core_axis_name="c", subcore_axis_name="s", num_cores=NC),
    out_type=[
        jax.ShapeDtypeStruct((B * 5 * D,), jnp.float32),
        jax.ShapeDtypeStruct((SEQ_N, 3 * D), jnp.float32),
        jax.ShapeDtypeStruct((B * 3 * D,), jnp.float32),
        jax.ShapeDtypeStruct((B * 3 * D,), jnp.float32),
    ],
    scratch_types=[
        pltpu.VMEM((CH,), jnp.int32),
        pltpu.VMEM((CH,), jnp.int32),
        pltpu.VMEM((CH, D), jnp.float32),
        pltpu.VMEM((CH, D), jnp.float32),
        pltpu.VMEM((SMALL_PER_W,), jnp.int32),
        pltpu.VMEM((SMALL_PER_W, D), jnp.float32),
        pltpu.VMEM((SMALL_PER_W * 5 * D,), jnp.float32),
        pltpu.SemaphoreType.DMA,
        pltpu.SemaphoreType.DMA,
        pltpu.SemaphoreType.DMA,
        pltpu.SemaphoreType.DMA,
    ],
    compiler_params=pltpu.CompilerParams(
        use_tc_tiling_on_sc=False, needs_layout_passes=False),
)(_sc_body)


def _relayout_body(uf, tf, sf, user_o, tseq_o, tside_o,
                   flat_v, big5, big3a, big3b):
  wid = lax.axis_index("s") * NC + lax.axis_index("c")
  lane = lax.broadcasted_iota(jnp.int32, (16,), 0)
  sbase = wid * SMALL_PER_W
  for src, out, big, w in ((uf, user_o, big5, 5 * D),
                           (tf, tseq_o, big3a, 3 * D),
                           (sf, tside_o, big3b, 3 * D)):
    n = SMALL_PER_W * w
    pltpu.sync_copy(src.at[pl.ds(sbase * w, n)], flat_v.at[pl.ds(0, n)])

    def go(g, _, big=big, w=w):
      q = g * 16 + lane
      r = q // w
      cc = q - w * r
      v = flat_v[pl.ds(g * 16, 16)]
      plsc.store_scatter(big, [r, cc], v)
      return 0
    lax.fori_loop(0, n // 16, go, 0)
    pltpu.sync_copy(big, out.at[pl.ds(sbase, SMALL_PER_W)])


_re_call = functools.partial(
    pl.kernel,
    mesh=plsc.VectorSubcoreMesh(
        core_axis_name="c", subcore_axis_name="s", num_cores=NC),
    out_type=[
        jax.ShapeDtypeStruct((B, 5 * D), jnp.float32),
        jax.ShapeDtypeStruct((B, 3 * D), jnp.float32),
        jax.ShapeDtypeStruct((B, 3 * D), jnp.float32),
    ],
    scratch_types=[
        pltpu.VMEM((SMALL_PER_W * 5 * D,), jnp.float32),
        pltpu.VMEM((SMALL_PER_W, 5 * D), jnp.float32),
        pltpu.VMEM((SMALL_PER_W, 3 * D), jnp.float32),
        pltpu.VMEM((SMALL_PER_W, 3 * D), jnp.float32),
    ],
    compiler_params=pltpu.CompilerParams(needs_layout_passes=False),
)(_relayout_body)


@jax.jit
def kernel(dense_inputs, target_user_side, seq_inputs, target_item_seq,
           target_item_side,
           W_seq0, W_seq1, W_seq2,
           W_user0, W_user1, W_user2, W_user3, W_user4,
           W_item0, W_item1, W_item2):
  del dense_inputs
  (s0, s1, s2, mask_i,
   u0, u1, u2, u3, u4, ts0, ts1, ts2, ti0, ti1, ti2) = _ex_call(
       seq_inputs, target_user_side, target_item_seq, target_item_side)

  user_f, seq_embed, tseq_f, tside_f = _sc_call(
      s0, s1, s2, W_seq0, W_seq1, W_seq2,
      u0, u1, u2, u3, u4,
      W_user0, W_user1, W_user2, W_user3, W_user4,
      ts0, ts1, ts2,
      ti0, ti1, ti2, W_item0, W_item1, W_item2)

  user_side, tseq, tside = _re_call(user_f, tseq_f, tside_f)
  mask_bool = mask_i.astype(jnp.bool_)
  return (mask_bool, user_side, seq_embed.reshape(B, HIST, 3 * D),
          tseq, tside)


# speedup vs baseline: 1.1029x; 1.1029x over previous
import functools

import jax
import jax.numpy as jnp
from jax import lax
from jax.experimental import pallas as pl
from jax.experimental.pallas import tpu as pltpu
from jax.experimental.pallas import tpu_sc as plsc

B = 4096
HIST = 200
D = 16

NC = 2
NS = 16
NW = NC * NS

SEQ_N = B * HIST
SEQ_PER_W = SEQ_N // NW
CH = 1280
NCH = SEQ_PER_W // CH
NPAIR = NCH // 2
IPG = 128
GPC = CH // IPG
SMALL_PER_W = B // NW


def _sc_body(s0, s1, s2, wseq0, wseq1, wseq2,
             u0, u1, u2, u3, u4, wu0, wu1, wu2, wu3, wu4,
             ts0, ts1, ts2,
             ti0, ti1, ti2, wi0, wi1, wi2,
             mask_o, user_o, seq_o, tseq_o, tside_o,
             idx0, idx1, m0, m1, rows0, rows1,
             sidx_v, srows_v,
             gsem, wsem0, wsem1, ssem):
  wid = lax.axis_index("s") * NC + lax.axis_index("c")
  base = wid * SEQ_PER_W

  sidx = [s0, s1, s2]
  wseq = [wseq0, wseq1, wseq2]
  idxs = [idx0, idx1]
  masks = [m0, m1]
  rows = [rows0, rows1]
  wsems = [wsem0, wsem1]

  def fire(t, p):
    def go(j, _):
      pltpu.async_copy(
          wseq[t].at[idxs[p].at[pl.ds(j * IPG, IPG)]],
          rows[p].at[pl.ds(j * IPG, IPG)], gsem)
      return 0
    lax.fori_loop(0, GPC, go, 0)

  def drain():
    def go(j, _):
      pltpu.make_async_copy(
          wseq[0].at[idx0.at[pl.ds(0, IPG)]],
          rows0.at[pl.ds(0, IPG)], gsem).wait()
      return 0
    lax.fori_loop(0, GPC, go, 0)

  def wait_write(p):
    pltpu.make_async_copy(
        rows[p], seq_o.at[pl.ds(0, CH), pl.ds(0, 16)], wsems[p]).wait()

  def pair_body(k, _):
    for half in range(2):
      c = 2 * k + half
      pos = base + c * CH
      for t in range(3):
        stage = 3 * half + t
        p = stage % 2
        pltpu.sync_copy(sidx[t].at[pl.ds(pos, CH)], idxs[p])
        if stage < 2:
          @pl.when(k > 0)
          def _():
            wait_write(p)
        else:
          wait_write(p)
        fire(t, p)
        if t == 0:
          def mask_body(j, _):
            v = idxs[p][pl.ds(j * 16, 16)]
            masks[p][pl.ds(j * 16, 16)] = jnp.where(
                v != 0, jnp.int32(1), jnp.int32(0))
            return 0
          lax.fori_loop(0, CH // 16, mask_body, 0)
          pltpu.sync_copy(masks[p], mask_o.at[pl.ds(pos, CH)])
        drain()
        pltpu.async_copy(
            rows[p], seq_o.at[pl.ds(pos, CH), pl.ds(16 * t, 16)], wsems[p])
    return 0

  lax.fori_loop(0, NPAIR, pair_body, 0)
  wait_write(0)
  wait_write(1)

  sbase = wid * SMALL_PER_W
  small = (
      [(([u0, u1, u2, u3, u4])[i], ([wu0, wu1, wu2, wu3, wu4])[i], user_o, i)
       for i in range(5)]
      + [(([ts0, ts1, ts2])[i], wseq[i], tseq_o, i) for i in range(3)]
      + [(([ti0, ti1, ti2])[i], ([wi0, wi1, wi2])[i], tside_o, i)
         for i in range(3)]
  )
  for idx_hbm, table, out, col in small:
    pltpu.sync_copy(idx_hbm.at[pl.ds(sbase, SMALL_PER_W)], sidx_v)
    pltpu.async_copy(table.at[sidx_v], srows_v, ssem).wait()
    pltpu.sync_copy(
        srows_v, out.at[pl.ds(sbase, SMALL_PER_W), pl.ds(16 * col, 16)])


_sc_call = functools.partial(
    pl.kernel,
    mesh=plsc.VectorSubcoreMesh(
        core_axis_name="c", subcore_axis_name="s", num_cores=NC),
    out_type=[
        jax.ShapeDtypeStruct((SEQ_N,), jnp.int32),
        jax.ShapeDtypeStruct((B, 5 * D), jnp.float32),
        jax.ShapeDtypeStruct((SEQ_N, 3 * D), jnp.float32),
        jax.ShapeDtypeStruct((B, 3 * D), jnp.float32),
        jax.ShapeDtypeStruct((B, 3 * D), jnp.float32),
    ],
    scratch_types=[
        pltpu.VMEM((CH,), jnp.int32),
        pltpu.VMEM((CH,), jnp.int32),
        pltpu.VMEM((CH,), jnp.int32),
        pltpu.VMEM((CH,), jnp.int32),
        pltpu.VMEM((CH, D), jnp.float32),
        pltpu.VMEM((CH, D), jnp.float32),
        pltpu.VMEM((SMALL_PER_W,), jnp.int32),
        pltpu.VMEM((SMALL_PER_W, D), jnp.float32),
        pltpu.SemaphoreType.DMA,
        pltpu.SemaphoreType.DMA,
        pltpu.SemaphoreType.DMA,
        pltpu.SemaphoreType.DMA,
    ],
    compiler_params=pltpu.CompilerParams(
        use_tc_tiling_on_sc=False, needs_layout_passes=False),
)(_sc_body)


@jax.jit
def kernel(dense_inputs, target_user_side, seq_inputs, target_item_seq,
           target_item_side,
           W_seq0, W_seq1, W_seq2,
           W_user0, W_user1, W_user2, W_user3, W_user4,
           W_item0, W_item1, W_item2):
  del dense_inputs
  s = [seq_inputs[:, :, i].reshape(-1) for i in range(3)]
  u = [target_user_side[:, i] for i in range(5)]
  ts = [target_item_seq[:, i] for i in range(3)]
  ti = [target_item_side[:, i] for i in range(3)]

  mask_i, user_side, seq_embed, tseq, tside = _sc_call(
      s[0], s[1], s[2], W_seq0, W_seq1, W_seq2,
      u[0], u[1], u[2], u[3], u[4],
      W_user0, W_user1, W_user2, W_user3, W_user4,
      ts[0], ts[1], ts[2],
      ti[0], ti[1], ti[2], W_item0, W_item1, W_item2)

  mask_bool = mask_i.reshape(B, HIST).astype(jnp.bool_)
  return (mask_bool, user_side, seq_embed.reshape(B, HIST, 3 * D),
          tseq, tside)


# speedup vs baseline: 1.1201x; 1.0156x over previous
import functools

import jax
import jax.numpy as jnp
from jax import lax
from jax.experimental import pallas as pl
from jax.experimental.pallas import tpu as pltpu
from jax.experimental.pallas import tpu_sc as plsc

B = 4096
HIST = 200
D = 16

NC = 2
NS = 16
NW = NC * NS

SEQ_N = B * HIST
SEQ_PER_W = SEQ_N // NW
CH = 2560
NCH = SEQ_PER_W // CH
NPAIR = NCH // 2
IPG = 128
GPC = CH // IPG
SMALL_PER_W = B // NW


def _sc_body(s0, s1, s2, wseq0, wseq1, wseq2,
             u0, u1, u2, u3, u4, wu0, wu1, wu2, wu3, wu4,
             ts0, ts1, ts2,
             ti0, ti1, ti2, wi0, wi1, wi2,
             mask_o, user_o, seq_o, tseq_o, tside_o,
             idx0, idx1, m0, m1, rows0, rows1,
             sidx_v, srows_v,
             gsem, wsem0, wsem1, ssem, isem0, isem1):
  wid = lax.axis_index("s") * NC + lax.axis_index("c")
  base = wid * SEQ_PER_W

  sidx = [s0, s1, s2]
  wseq = [wseq0, wseq1, wseq2]
  idxs = [idx0, idx1]
  masks = [m0, m1]
  rows = [rows0, rows1]
  wsems = [wsem0, wsem1]
  isems = [isem0, isem1]

  def fire(t, p):
    def go(j, _):
      pltpu.async_copy(
          wseq[t].at[idxs[p].at[pl.ds(j * IPG, IPG)]],
          rows[p].at[pl.ds(j * IPG, IPG)], gsem)
      return 0
    lax.fori_loop(0, GPC, go, 0)

  def drain():
    def go(j, _):
      pltpu.make_async_copy(
          wseq[0].at[idx0.at[pl.ds(0, IPG)]],
          rows0.at[pl.ds(0, IPG)], gsem).wait()
      return 0
    lax.fori_loop(0, GPC, go, 0)

  def wait_write(p):
    pltpu.make_async_copy(
        rows[p], seq_o.at[pl.ds(0, CH), pl.ds(0, 16)], wsems[p]).wait()

  pltpu.async_copy(s0.at[pl.ds(base, CH)], idx0, isem0)

  def pair_body(k, _):
    for half in range(2):
      c = 2 * k + half
      pos = base + c * CH
      for t in range(3):
        stage = 3 * half + t
        p = stage % 2
        pltpu.make_async_copy(
            s0.at[pl.ds(0, CH)], idxs[p], isems[p]).wait()
        if stage < 2:
          @pl.when(k > 0)
          def _():
            wait_write(p)
        else:
          wait_write(p)
        fire(t, p)
        if t < 2:
          pltpu.async_copy(
              sidx[t + 1].at[pl.ds(pos, CH)], idxs[1 - p], isems[1 - p])
        elif half == 0:
          pltpu.async_copy(
              s0.at[pl.ds(pos + CH, CH)], idxs[1 - p], isems[1 - p])
        else:
          @pl.when(k < NPAIR - 1)
          def _():
            pltpu.async_copy(
                s0.at[pl.ds(pos + CH, CH)], idxs[1 - p], isems[1 - p])
        if t == 0:
          def mask_body(j, _):
            v = idxs[p][pl.ds(j * 16, 16)]
            masks[p][pl.ds(j * 16, 16)] = jnp.where(
                v != 0, jnp.int32(1), jnp.int32(0))
            return 0
          lax.fori_loop(0, CH // 16, mask_body, 0)
          pltpu.sync_copy(masks[p], mask_o.at[pl.ds(pos, CH)])
        drain()
        pltpu.async_copy(
            rows[p], seq_o.at[pl.ds(pos, CH), pl.ds(16 * t, 16)], wsems[p])
    return 0

  lax.fori_loop(0, NPAIR, pair_body, 0)
  wait_write(0)
  wait_write(1)

  sbase = wid * SMALL_PER_W
  small = (
      [(([u0, u1, u2, u3, u4])[i], ([wu0, wu1, wu2, wu3, wu4])[i], user_o, i)
       for i in range(5)]
      + [(([ts0, ts1, ts2])[i], wseq[i], tseq_o, i) for i in range(3)]
      + [(([ti0, ti1, ti2])[i], ([wi0, wi1, wi2])[i], tside_o, i)
         for i in range(3)]
  )
  for idx_hbm, table, out, col in small:
    pltpu.sync_copy(idx_hbm.at[pl.ds(sbase, SMALL_PER_W)], sidx_v)
    pltpu.async_copy(table.at[sidx_v], srows_v, ssem).wait()
    pltpu.sync_copy(
        srows_v, out.at[pl.ds(sbase, SMALL_PER_W), pl.ds(16 * col, 16)])


_sc_call = functools.partial(
    pl.kernel,
    mesh=plsc.VectorSubcoreMesh(
        core_axis_name="c", subcore_axis_name="s", num_cores=NC),
    out_type=[
        jax.ShapeDtypeStruct((SEQ_N,), jnp.int32),
        jax.ShapeDtypeStruct((B, 5 * D), jnp.float32),
        jax.ShapeDtypeStruct((SEQ_N, 3 * D), jnp.float32),
        jax.ShapeDtypeStruct((B, 3 * D), jnp.float32),
        jax.ShapeDtypeStruct((B, 3 * D), jnp.float32),
    ],
    scratch_types=[
        pltpu.VMEM((CH,), jnp.int32),
        pltpu.VMEM((CH,), jnp.int32),
        pltpu.VMEM((CH,), jnp.int32),
        pltpu.VMEM((CH,), jnp.int32),
        pltpu.VMEM((CH, D), jnp.float32),
        pltpu.VMEM((CH, D), jnp.float32),
        pltpu.VMEM((SMALL_PER_W,), jnp.int32),
        pltpu.VMEM((SMALL_PER_W, D), jnp.float32),
        pltpu.SemaphoreType.DMA,
        pltpu.SemaphoreType.DMA,
        pltpu.SemaphoreType.DMA,
        pltpu.SemaphoreType.DMA,
        pltpu.SemaphoreType.DMA,
        pltpu.SemaphoreType.DMA,
    ],
    compiler_params=pltpu.CompilerParams(
        use_tc_tiling_on_sc=False, needs_layout_passes=False),
)(_sc_body)


@jax.jit
def kernel(dense_inputs, target_user_side, seq_inputs, target_item_seq,
           target_item_side,
           W_seq0, W_seq1, W_seq2,
           W_user0, W_user1, W_user2, W_user3, W_user4,
           W_item0, W_item1, W_item2):
  del dense_inputs
  s = [seq_inputs[:, :, i].reshape(-1) for i in range(3)]
  u = [target_user_side[:, i] for i in range(5)]
  ts = [target_item_seq[:, i] for i in range(3)]
  ti = [target_item_side[:, i] for i in range(3)]

  mask_i, user_side, seq_embed, tseq, tside = _sc_call(
      s[0], s[1], s[2], W_seq0, W_seq1, W_seq2,
      u[0], u[1], u[2], u[3], u[4],
      W_user0, W_user1, W_user2, W_user3, W_user4,
      ts[0], ts[1], ts[2],
      ti[0], ti[1], ti[2], W_item0, W_item1, W_item2)

  mask_bool = mask_i.reshape(B, HIST).astype(jnp.bool_)
  return (mask_bool, user_side, seq_embed.reshape(B, HIST, 3 * D),
          tseq, tside)


# speedup vs baseline: 1.1796x; 1.0532x over previous
import functools

import jax
import jax.numpy as jnp
from jax import lax
from jax.experimental import pallas as pl
from jax.experimental.pallas import tpu as pltpu
from jax.experimental.pallas import tpu_sc as plsc

B = 4096
HIST = 200
D = 16

NC = 2
NS = 16
NW = NC * NS

SEQ_N = B * HIST
SEQ_PER_W = SEQ_N // NW
CH = 2560
NCH = SEQ_PER_W // CH
NPAIR = NCH // 2
IPG = 128
GPC = CH // IPG
SMALL_PER_W = B // NW


def _sc_body(s0, s1, s2, wseq0, wseq1, wseq2,
             u0, u1, u2, u3, u4, wu0, wu1, wu2, wu3, wu4,
             ts0, ts1, ts2,
             ti0, ti1, ti2, wi0, wi1, wi2,
             mask_o, user_o, seq_o, tseq_o, tside_o,
             idx0, idx1, m0, m1, rows0, rows1,
             sidxs, srows,
             gsem, wsem0, wsem1, ssem, isem0, isem1):
  wid = lax.axis_index("s") * NC + lax.axis_index("c")
  base = wid * SEQ_PER_W
  sbase = wid * SMALL_PER_W

  small = (
      [(([u0, u1, u2, u3, u4])[i], ([wu0, wu1, wu2, wu3, wu4])[i], user_o, i)
       for i in range(5)]
      + [(([ts0, ts1, ts2])[i], ([wseq0, wseq1, wseq2])[i], tseq_o, i)
         for i in range(3)]
      + [(([ti0, ti1, ti2])[i], ([wi0, wi1, wi2])[i], tside_o, i)
         for i in range(3)]
  )
  for i, (idx_hbm, _, _, _) in enumerate(small):
    pltpu.async_copy(idx_hbm.at[pl.ds(sbase, SMALL_PER_W)], sidxs[i], ssem)

  sidx = [s0, s1, s2]
  wseq = [wseq0, wseq1, wseq2]
  idxs = [idx0, idx1]
  masks = [m0, m1]
  rows = [rows0, rows1]
  wsems = [wsem0, wsem1]
  isems = [isem0, isem1]

  def fire(t, p):
    def go(j, _):
      pltpu.async_copy(
          wseq[t].at[idxs[p].at[pl.ds(j * IPG, IPG)]],
          rows[p].at[pl.ds(j * IPG, IPG)], gsem)
      return 0
    lax.fori_loop(0, GPC, go, 0)

  def drain():
    def go(j, _):
      pltpu.make_async_copy(
          wseq[0].at[idx0.at[pl.ds(0, IPG)]],
          rows0.at[pl.ds(0, IPG)], gsem).wait()
      return 0
    lax.fori_loop(0, GPC, go, 0)

  def wait_write(p):
    pltpu.make_async_copy(
        rows[p], seq_o.at[pl.ds(0, CH), pl.ds(0, 16)], wsems[p]).wait()

  pltpu.async_copy(s0.at[pl.ds(base, CH)], idx0, isem0)

  def pair_body(k, _):
    for half in range(2):
      c = 2 * k + half
      pos = base + c * CH
      for t in range(3):
        stage = 3 * half + t
        p = stage % 2
        pltpu.make_async_copy(
            s0.at[pl.ds(0, CH)], idxs[p], isems[p]).wait()
        if stage < 2:
          @pl.when(k > 0)
          def _():
            wait_write(p)
        else:
          wait_write(p)
        fire(t, p)
        if t < 2:
          pltpu.async_copy(
              sidx[t + 1].at[pl.ds(pos, CH)], idxs[1 - p], isems[1 - p])
        elif half == 0:
          pltpu.async_copy(
              s0.at[pl.ds(pos + CH, CH)], idxs[1 - p], isems[1 - p])
        else:
          @pl.when(k < NPAIR - 1)
          def _():
            pltpu.async_copy(
                s0.at[pl.ds(pos + CH, CH)], idxs[1 - p], isems[1 - p])
        if t == 0:
          def mask_body(j, _):
            v = idxs[p][pl.ds(j * 16, 16)]
            masks[p][pl.ds(j * 16, 16)] = jnp.where(
                v != 0, jnp.int32(1), jnp.int32(0))
            return 0
          lax.fori_loop(0, CH // 16, mask_body, 0)
          pltpu.sync_copy(masks[p], mask_o.at[pl.ds(pos, CH)])
        drain()
        pltpu.async_copy(
            rows[p], seq_o.at[pl.ds(pos, CH), pl.ds(16 * t, 16)], wsems[p])
    return 0

  lax.fori_loop(0, NPAIR, pair_body, 0)
  wait_write(0)
  wait_write(1)

  for i, (idx_hbm, _, _, _) in enumerate(small):
    pltpu.make_async_copy(
        idx_hbm.at[pl.ds(sbase, SMALL_PER_W)], sidxs[i], ssem).wait()
  for i, (_, table, _, _) in enumerate(small):
    pltpu.async_copy(table.at[sidxs[i]], srows[i], ssem)
  for i, (_, table, _, _) in enumerate(small):
    pltpu.make_async_copy(table.at[sidxs[i]], srows[i], ssem).wait()
  for i, (_, _, out, col) in enumerate(small):
    pltpu.async_copy(
        srows[i], out.at[pl.ds(sbase, SMALL_PER_W), pl.ds(16 * col, 16)],
        wsem0)
  for i, (_, _, out, col) in enumerate(small):
    pltpu.make_async_copy(
        srows[i], out.at[pl.ds(sbase, SMALL_PER_W), pl.ds(16 * col, 16)],
        wsem0).wait()


_sc_call = functools.partial(
    pl.kernel,
    mesh=plsc.VectorSubcoreMesh(
        core_axis_name="c", subcore_axis_name="s", num_cores=NC),
    out_type=[
        jax.ShapeDtypeStruct((SEQ_N,), jnp.int32),
        jax.ShapeDtypeStruct((B, 5 * D), jnp.float32),
        jax.ShapeDtypeStruct((SEQ_N, 3 * D), jnp.float32),
        jax.ShapeDtypeStruct((B, 3 * D), jnp.float32),
        jax.ShapeDtypeStruct((B, 3 * D), jnp.float32),
    ],
    scratch_types=[
        pltpu.VMEM((CH,), jnp.int32),
        pltpu.VMEM((CH,), jnp.int32),
        pltpu.VMEM((CH,), jnp.int32),
        pltpu.VMEM((CH,), jnp.int32),
        pltpu.VMEM((CH, D), jnp.float32),
        pltpu.VMEM((CH, D), jnp.float32),
        [pltpu.VMEM((SMALL_PER_W,), jnp.int32) for _ in range(11)],
        [pltpu.VMEM((SMALL_PER_W, D), jnp.float32) for _ in range(11)],
        pltpu.SemaphoreType.DMA,
        pltpu.SemaphoreType.DMA,
        pltpu.SemaphoreType.DMA,
        pltpu.SemaphoreType.DMA,
        pltpu.SemaphoreType.DMA,
        pltpu.SemaphoreType.DMA,
    ],
    compiler_params=pltpu.CompilerParams(
        use_tc_tiling_on_sc=False, needs_layout_passes=False),
)(_sc_body)


@jax.jit
def kernel(dense_inputs, target_user_side, seq_inputs, target_item_seq,
           target_item_side,
           W_seq0, W_seq1, W_seq2,
           W_user0, W_user1, W_user2, W_user3, W_user4,
           W_item0, W_item1, W_item2):
  del dense_inputs
  s = [seq_inputs[:, :, i].reshape(-1) for i in range(3)]
  u = [target_user_side[:, i] for i in range(5)]
  ts = [target_item_seq[:, i] for i in range(3)]
  ti = [target_item_side[:, i] for i in range(3)]

  mask_i, user_side, seq_embed, tseq, tside = _sc_call(
      s[0], s[1], s[2], W_seq0, W_seq1, W_seq2,
      u[0], u[1], u[2], u[3], u[4],
      W_user0, W_user1, W_user2, W_user3, W_user4,
      ts[0], ts[1], ts[2],
      ti[0], ti[1], ti[2], W_item0, W_item1, W_item2)

  mask_bool = mask_i.reshape(B, HIST).astype(jnp.bool_)
  return (mask_bool, user_side, seq_embed.reshape(B, HIST, 3 * D),
          tseq, tside)
